# Initial kernel scaffold; baseline (speedup 1.0000x reference)
import jax
import jax.numpy as jnp
from jax.experimental import pallas as pl


def kernel(x, edge_index, Wl0, bl0, Wr0, gamma0, beta0, Wls, bls, Wrs, gammas, betas, Wh, bh):
    raise NotImplementedError("write your pallas kernel here")


# speedup vs baseline: 4.6092x; 4.6092x over previous
import functools

import jax
import jax.numpy as jnp
from jax import lax
from jax.experimental import pallas as pl
from jax.experimental.pallas import tpu as pltpu
from jax.experimental.pallas import tpu_sc as plsc

NNODE = 50000
NEDGE = 800000
IN_D = 8
HID = 64
HH = 32
NL = 8
ODIM = 200
BNEPS = 1e-5

NACC = 51200
FLUSH = 128
EPAD = 819200
EROWS = 6400
IDXB = 8
INNER = 4
OUTER = 50

BLK = 2000
NB = 25


def _sc_body(h_hbm, src_hbm, dst_hbm, out_hbm, src_v, dst_v, rows_v,
             acc_sh, sem):
    c = lax.axis_index("c")
    s = lax.axis_index("s")

    def zrow(i, carry):
        rows_v[0, i, pl.ds(0, 16)] = jnp.zeros((16,), jnp.float32)
        rows_v[0, i, pl.ds(16, 16)] = jnp.zeros((16,), jnp.float32)
        return carry
    lax.fori_loop(0, FLUSH, zrow, 0)
    base = s * (25 * FLUSH)

    def zero_step(q, carry):
        pltpu.sync_copy(rows_v.at[0], acc_sh.at[pl.ds(base + q * FLUSH, FLUSH)])
        return carry
    lax.fori_loop(0, 25, zero_step, 0)
    plsc.subcore_barrier()

    row0 = s * (OUTER * IDXB)

    def edge_step(o, carry):
        rowbase = row0 + o * IDXB
        pltpu.sync_copy(src_hbm.at[pl.ds(rowbase, IDXB)], src_v)
        pltpu.sync_copy(dst_hbm.at[pl.ds(rowbase, IDXB)], dst_v)

        def half(t, carry2):
            cps = []
            for j in range(INNER):
                cps.append(pltpu.async_copy(
                    h_hbm.at[c].at[src_v.at[t * INNER + j]], rows_v.at[j],
                    sem))
            for j in range(INNER):
                cps[j].wait()

            def scat(j, carry3):
                pltpu.sync_copy(rows_v.at[j],
                                acc_sh.at[dst_v.at[t * INNER + j]], add=True)
                return carry3
            lax.fori_loop(0, INNER, scat, 0)
            return carry2
        lax.fori_loop(0, IDXB // INNER, half, 0)
        return carry
    lax.fori_loop(0, OUTER, edge_step, 0)
    plsc.subcore_barrier()

    def flush_step(q, carry):
        pltpu.sync_copy(acc_sh.at[pl.ds(base + q * FLUSH, FLUSH)], rows_v.at[0])
        pltpu.sync_copy(rows_v.at[0],
                        out_hbm.at[c].at[pl.ds(base + q * FLUSH, FLUSH)])
        return carry
    lax.fori_loop(0, 25, flush_step, 0)


def _sc_agg(table, src2d, dst2d):
    mesh = plsc.VectorSubcoreMesh(core_axis_name="c", subcore_axis_name="s")
    return pl.kernel(
        _sc_body,
        out_type=jax.ShapeDtypeStruct((2, NACC, HH), jnp.float32),
        mesh=mesh,
        scratch_types=[
            pltpu.VMEM((IDXB, 128), jnp.int32),
            pltpu.VMEM((IDXB, 128), jnp.int32),
            pltpu.VMEM((INNER, 128, HH), jnp.float32),
            pltpu.VMEM_SHARED((NACC, HH), jnp.float32),
            pltpu.SemaphoreType.DMA,
        ],
        compiler_params=pltpu.CompilerParams(use_tc_tiling_on_sc=False),
    )(table, src2d, dst2d)


def _tc_layer_body(acc_ref, h_ref, wl_ref, wr_ref, g_ref, be_ref, out_ref,
                   z_ref, st_ref):
    p = pl.program_id(0)
    b = pl.program_id(1)

    @pl.when(p == 0)
    def _():
        agg = jnp.concatenate([acc_ref[0], acc_ref[1]], axis=1)
        h = jnp.concatenate([h_ref[0], h_ref[1]], axis=1)
        z = (lax.dot_general(agg, wl_ref[...], (((1,), (1,)), ((), ())),
                             preferred_element_type=jnp.float32)
             + lax.dot_general(h, wr_ref[...], (((1,), (1,)), ((), ())),
                               preferred_element_type=jnp.float32))
        z_ref[pl.ds(b * BLK, BLK), :] = z
        s0 = jnp.sum(z, axis=0, keepdims=True)
        s1 = jnp.sum(z * z, axis=0, keepdims=True)

        @pl.when(b == 0)
        def _():
            st_ref[0:1, :] = s0
            st_ref[1:2, :] = s1

        @pl.when(b > 0)
        def _():
            st_ref[0:1, :] = st_ref[0:1, :] + s0
            st_ref[1:2, :] = st_ref[1:2, :] + s1

    @pl.when(p == 1)
    def _():
        mu = st_ref[0:1, :] * (1.0 / NNODE)
        var = st_ref[1:2, :] * (1.0 / NNODE) - mu * mu
        inv = g_ref[...] * lax.rsqrt(var + BNEPS)
        z = z_ref[pl.ds(b * BLK, BLK), :]
        hn = jnp.tanh(z * inv + (be_ref[...] - mu * inv))
        out_ref[0] = hn[:, :HH]
        out_ref[1] = hn[:, HH:]


def _tc_layer(acc, h, Wl, Wr, gamma, beta):
    return pl.pallas_call(
        _tc_layer_body,
        grid=(2, NB),
        in_specs=[
            pl.BlockSpec((2, BLK, HH), lambda p, b: (0, b, 0)),
            pl.BlockSpec((2, BLK, HH), lambda p, b: (0, b, 0)),
            pl.BlockSpec((HID, HID), lambda p, b: (0, 0)),
            pl.BlockSpec((HID, HID), lambda p, b: (0, 0)),
            pl.BlockSpec((1, HID), lambda p, b: (0, 0)),
            pl.BlockSpec((1, HID), lambda p, b: (0, 0)),
        ],
        out_specs=pl.BlockSpec((2, BLK, HH), lambda p, b: (0, b, 0)),
        out_shape=jax.ShapeDtypeStruct((2, NACC, HH), jnp.float32),
        scratch_shapes=[
            pltpu.VMEM((NNODE, HID), jnp.float32),
            pltpu.VMEM((2, HID), jnp.float32),
        ],
    )(acc, h, Wl, Wr, gamma, beta)


def _tc_head_body(h_ref, wh_ref, bh_ref, out_ref):
    h = jnp.concatenate([h_ref[0], h_ref[1]], axis=1)
    out_ref[...] = lax.dot_general(
        h, wh_ref[...], (((1,), (1,)), ((), ())),
        preferred_element_type=jnp.float32) + bh_ref[...]


def _tc_head(h, Wh, bh):
    return pl.pallas_call(
        _tc_head_body,
        grid=(NB,),
        in_specs=[
            pl.BlockSpec((2, BLK, HH), lambda b: (0, b, 0)),
            pl.BlockSpec((ODIM, HID), lambda b: (0, 0)),
            pl.BlockSpec((1, ODIM), lambda b: (0, 0)),
        ],
        out_specs=pl.BlockSpec((BLK, ODIM), lambda b: (b, 0)),
        out_shape=jax.ShapeDtypeStruct((NNODE, ODIM), jnp.float32),
    )(h, Wh, bh)


def kernel(x, edge_index, Wl0, bl0, Wr0, gamma0, beta0, Wls, bls, Wrs,
           gammas, betas, Wh, bh):
    del bl0, bls
    src = edge_index[0]
    dst = edge_index[1]
    npad = EPAD - NEDGE
    src2d = jnp.concatenate(
        [src, jnp.zeros((npad,), jnp.int32)]).reshape(EROWS, 128)
    dst2d = jnp.concatenate(
        [dst, jnp.full((npad,), NNODE, jnp.int32)]).reshape(EROWS, 128)

    h0 = jnp.zeros((2, NACC, HH), jnp.float32).at[0, :NNODE, :IN_D].set(x)
    wpad = jnp.zeros((HID, HID - IN_D), jnp.float32)
    Wl_all = jnp.concatenate(
        [jnp.concatenate([Wl0, wpad], axis=1)[None], Wls], axis=0)
    Wr_all = jnp.concatenate(
        [jnp.concatenate([Wr0, wpad], axis=1)[None], Wrs], axis=0)
    g_all = jnp.concatenate([gamma0[None], gammas], axis=0)
    b_all = jnp.concatenate([beta0[None], betas], axis=0)

    def layer(h, ws):
        Wl, Wr, g, be = ws
        acc = _sc_agg(h, src2d, dst2d)
        h2 = _tc_layer(acc, h, Wl, Wr, g.reshape(1, HID), be.reshape(1, HID))
        return h2, None

    h, _ = lax.scan(layer, h0, (Wl_all, Wr_all, g_all, b_all))
    return _tc_head(h, Wh, bh.reshape(1, ODIM))


# speedup vs baseline: 4.8170x; 1.0451x over previous
import functools

import jax
import jax.numpy as jnp
from jax import lax
from jax.experimental import pallas as pl
from jax.experimental.pallas import tpu as pltpu
from jax.experimental.pallas import tpu_sc as plsc

NNODE = 50000
NEDGE = 800000
IN_D = 8
HID = 64
HH = 32
NL = 8
ODIM = 200
BNEPS = 1e-5

NACC = 51200
FLUSH = 128
EPAD = 819200
EROWS = 6400
IDXB = 40
INNER = 4
OUTER = 10

BLK = 2000
NB = 25


def _sc_body(h_hbm, src_hbm, dst_hbm, out_hbm, src_v, dst_v, rows_v,
             acc_sh, sem):
    c = lax.axis_index("c")
    s = lax.axis_index("s")

    def zrow(i, carry):
        rows_v[0, i, pl.ds(0, 16)] = jnp.zeros((16,), jnp.float32)
        rows_v[0, i, pl.ds(16, 16)] = jnp.zeros((16,), jnp.float32)
        return carry
    lax.fori_loop(0, FLUSH, zrow, 0)
    base = s * (25 * FLUSH)

    def zero_step(q, carry):
        pltpu.sync_copy(rows_v.at[0], acc_sh.at[pl.ds(base + q * FLUSH, FLUSH)])
        return carry
    lax.fori_loop(0, 25, zero_step, 0)
    plsc.subcore_barrier()

    row0 = s * (OUTER * IDXB)

    def edge_step(o, carry):
        rowbase = row0 + o * IDXB
        pltpu.sync_copy(src_hbm.at[pl.ds(rowbase, IDXB)], src_v)
        pltpu.sync_copy(dst_hbm.at[pl.ds(rowbase, IDXB)], dst_v)

        def half(t, carry2):
            cps = []
            for j in range(INNER):
                cps.append(pltpu.async_copy(
                    h_hbm.at[c].at[src_v.at[t * INNER + j]], rows_v.at[j],
                    sem))
            for j in range(INNER):
                cps[j].wait()

            def scat(j, carry3):
                pltpu.sync_copy(rows_v.at[j],
                                acc_sh.at[dst_v.at[t * INNER + j]], add=True)
                return carry3
            lax.fori_loop(0, INNER, scat, 0)
            return carry2
        lax.fori_loop(0, IDXB // INNER, half, 0)
        return carry
    lax.fori_loop(0, OUTER, edge_step, 0)
    plsc.subcore_barrier()

    def flush_step(q, carry):
        pltpu.sync_copy(acc_sh.at[pl.ds(base + q * FLUSH, FLUSH)], rows_v.at[0])
        pltpu.sync_copy(rows_v.at[0],
                        out_hbm.at[c].at[pl.ds(base + q * FLUSH, FLUSH)])
        return carry
    lax.fori_loop(0, 25, flush_step, 0)


def _sc_agg(table, src2d, dst2d):
    mesh = plsc.VectorSubcoreMesh(core_axis_name="c", subcore_axis_name="s")
    return pl.kernel(
        _sc_body,
        out_type=jax.ShapeDtypeStruct((2, NACC, HH), jnp.float32),
        mesh=mesh,
        scratch_types=[
            pltpu.VMEM((IDXB, 128), jnp.int32),
            pltpu.VMEM((IDXB, 128), jnp.int32),
            pltpu.VMEM((INNER, 128, HH), jnp.float32),
            pltpu.VMEM_SHARED((NACC, HH), jnp.float32),
            pltpu.SemaphoreType.DMA,
        ],
        compiler_params=pltpu.CompilerParams(use_tc_tiling_on_sc=False),
    )(table, src2d, dst2d)


def _tc_layer_body(acc_ref, h_ref, wl_ref, wr_ref, g_ref, be_ref, out_ref,
                   z_ref, st_ref):
    p = pl.program_id(0)
    b = pl.program_id(1)

    @pl.when(p == 0)
    def _():
        agg = jnp.concatenate([acc_ref[0], acc_ref[1]], axis=1)
        h = jnp.concatenate([h_ref[0], h_ref[1]], axis=1)
        z = (lax.dot_general(agg, wl_ref[...], (((1,), (1,)), ((), ())),
                             preferred_element_type=jnp.float32)
             + lax.dot_general(h, wr_ref[...], (((1,), (1,)), ((), ())),
                               preferred_element_type=jnp.float32))
        z_ref[pl.ds(b * BLK, BLK), :] = z
        s0 = jnp.sum(z, axis=0, keepdims=True)
        s1 = jnp.sum(z * z, axis=0, keepdims=True)

        @pl.when(b == 0)
        def _():
            st_ref[0:1, :] = s0
            st_ref[1:2, :] = s1

        @pl.when(b > 0)
        def _():
            st_ref[0:1, :] = st_ref[0:1, :] + s0
            st_ref[1:2, :] = st_ref[1:2, :] + s1

    @pl.when(p == 1)
    def _():
        mu = st_ref[0:1, :] * (1.0 / NNODE)
        var = st_ref[1:2, :] * (1.0 / NNODE) - mu * mu
        inv = g_ref[...] * lax.rsqrt(var + BNEPS)
        z = z_ref[pl.ds(b * BLK, BLK), :]
        hn = jnp.tanh(z * inv + (be_ref[...] - mu * inv))
        out_ref[0] = hn[:, :HH]
        out_ref[1] = hn[:, HH:]


def _tc_layer(acc, h, Wl, Wr, gamma, beta):
    return pl.pallas_call(
        _tc_layer_body,
        grid=(2, NB),
        in_specs=[
            pl.BlockSpec((2, BLK, HH), lambda p, b: (0, b, 0)),
            pl.BlockSpec((2, BLK, HH), lambda p, b: (0, b, 0)),
            pl.BlockSpec((HID, HID), lambda p, b: (0, 0)),
            pl.BlockSpec((HID, HID), lambda p, b: (0, 0)),
            pl.BlockSpec((1, HID), lambda p, b: (0, 0)),
            pl.BlockSpec((1, HID), lambda p, b: (0, 0)),
        ],
        out_specs=pl.BlockSpec((2, BLK, HH), lambda p, b: (0, b, 0)),
        out_shape=jax.ShapeDtypeStruct((2, NACC, HH), jnp.float32),
        scratch_shapes=[
            pltpu.VMEM((NNODE, HID), jnp.float32),
            pltpu.VMEM((2, HID), jnp.float32),
        ],
    )(acc, h, Wl, Wr, gamma, beta)


def _tc_head_body(h_ref, wh_ref, bh_ref, out_ref):
    h = jnp.concatenate([h_ref[0], h_ref[1]], axis=1)
    out_ref[...] = lax.dot_general(
        h, wh_ref[...], (((1,), (1,)), ((), ())),
        preferred_element_type=jnp.float32) + bh_ref[...]


def _tc_head(h, Wh, bh):
    return pl.pallas_call(
        _tc_head_body,
        grid=(NB,),
        in_specs=[
            pl.BlockSpec((2, BLK, HH), lambda b: (0, b, 0)),
            pl.BlockSpec((ODIM, HID), lambda b: (0, 0)),
            pl.BlockSpec((1, ODIM), lambda b: (0, 0)),
        ],
        out_specs=pl.BlockSpec((BLK, ODIM), lambda b: (b, 0)),
        out_shape=jax.ShapeDtypeStruct((NNODE, ODIM), jnp.float32),
    )(h, Wh, bh)


def kernel(x, edge_index, Wl0, bl0, Wr0, gamma0, beta0, Wls, bls, Wrs,
           gammas, betas, Wh, bh):
    del bl0, bls
    src = edge_index[0]
    dst = edge_index[1]
    npad = EPAD - NEDGE
    src2d = jnp.concatenate(
        [src, jnp.zeros((npad,), jnp.int32)]).reshape(EROWS, 128)
    dst2d = jnp.concatenate(
        [dst, jnp.full((npad,), NNODE, jnp.int32)]).reshape(EROWS, 128)

    h0 = jnp.zeros((2, NACC, HH), jnp.float32).at[0, :NNODE, :IN_D].set(x)
    wpad = jnp.zeros((HID, HID - IN_D), jnp.float32)
    Wl_all = jnp.concatenate(
        [jnp.concatenate([Wl0, wpad], axis=1)[None], Wls], axis=0)
    Wr_all = jnp.concatenate(
        [jnp.concatenate([Wr0, wpad], axis=1)[None], Wrs], axis=0)
    g_all = jnp.concatenate([gamma0[None], gammas], axis=0)
    b_all = jnp.concatenate([beta0[None], betas], axis=0)

    def layer(h, ws):
        Wl, Wr, g, be = ws
        acc = _sc_agg(h, src2d, dst2d)
        h2 = _tc_layer(acc, h, Wl, Wr, g.reshape(1, HID), be.reshape(1, HID))
        return h2, None

    h, _ = lax.scan(layer, h0, (Wl_all, Wr_all, g_all, b_all))
    return _tc_head(h, Wh, bh.reshape(1, ODIM))


# speedup vs baseline: 5.3802x; 1.1169x over previous
import functools

import jax
import jax.numpy as jnp
from jax import lax
from jax.experimental import pallas as pl
from jax.experimental.pallas import tpu as pltpu
from jax.experimental.pallas import tpu_sc as plsc

NNODE = 50000
NEDGE = 800000
IN_D = 8
HID = 64
HH = 32
NL = 8
ODIM = 200
BNEPS = 1e-5

NACC = 51200
CH = 64
FLUSH = 64
EPAD = 819200
EROWS = 12800
IDXB = 80
NSLOT = 8
OUTER = 10
NBODY = 10

BLK = 2000
NB = 25


def _sc_body(h_hbm, src_hbm, dst_hbm, out_hbm, src_v, dst_v, rows_v,
             acc_sh, gsem, ssem):
    c = lax.axis_index("c")
    s = lax.axis_index("s")

    def zrow(i, carry):
        rows_v[0, i, pl.ds(0, 16)] = jnp.zeros((16,), jnp.float32)
        rows_v[0, i, pl.ds(16, 16)] = jnp.zeros((16,), jnp.float32)
        return carry
    lax.fori_loop(0, FLUSH, zrow, 0)
    base = s * (50 * FLUSH)

    def zero_step(q, carry):
        pltpu.sync_copy(rows_v.at[0], acc_sh.at[pl.ds(base + q * FLUSH, FLUSH)])
        return carry
    lax.fori_loop(0, 50, zero_step, 0)
    plsc.subcore_barrier()

    row0 = s * (OUTER * IDXB)

    def fire_gather(r, slot):
        pltpu.async_copy(h_hbm.at[c].at[src_v.at[r]], rows_v.at[slot], gsem)

    def drain_gather(slot):
        pltpu.make_async_copy(h_hbm.at[c, pl.ds(0, CH)], rows_v.at[slot],
                              gsem).wait()

    def fire_scatter(r, slot):
        pltpu.async_copy(rows_v.at[slot], acc_sh.at[dst_v.at[r]], ssem,
                         add=True)

    def drain_scatter(slot):
        pltpu.make_async_copy(h_hbm.at[c, pl.ds(0, CH)], rows_v.at[slot],
                              ssem).wait()

    def load_idx(o):
        rowbase = row0 + o * IDXB
        pltpu.sync_copy(src_hbm.at[pl.ds(rowbase, IDXB)], src_v)
        pltpu.sync_copy(dst_hbm.at[pl.ds(rowbase, IDXB)], dst_v)

    def body(r0, first):
        @pl.when(jnp.logical_not(first))
        def _():
            for j in range(4):
                drain_scatter(j)
        for j in range(4):
            fire_gather(r0 + j, j)

        @pl.when(jnp.logical_not(first))
        def _():
            for j in range(4):
                drain_scatter(4 + j)
        for j in range(4):
            fire_gather(r0 + 4 + j, 4 + j)
        for j in range(4):
            drain_gather(j)
        for j in range(4):
            fire_scatter(r0 + j, j)
        for j in range(4):
            drain_gather(4 + j)
        for j in range(4):
            fire_scatter(r0 + 4 + j, 4 + j)

    def edge_step(o, carry):
        load_idx(o)

        def body_step(t, carry2):
            body(t * 8, jnp.logical_and(o == 0, t == 0))
            return carry2
        lax.fori_loop(0, NBODY, body_step, 0)
        return carry
    lax.fori_loop(0, OUTER, edge_step, 0)
    for j in range(NSLOT):
        drain_scatter(j)
    plsc.subcore_barrier()

    def flush_step(q, carry):
        pltpu.sync_copy(acc_sh.at[pl.ds(base + q * FLUSH, FLUSH)], rows_v.at[0])
        pltpu.sync_copy(rows_v.at[0],
                        out_hbm.at[c].at[pl.ds(base + q * FLUSH, FLUSH)])
        return carry
    lax.fori_loop(0, 50, flush_step, 0)


def _sc_agg(table, src2d, dst2d):
    mesh = plsc.VectorSubcoreMesh(core_axis_name="c", subcore_axis_name="s")
    return pl.kernel(
        _sc_body,
        out_type=jax.ShapeDtypeStruct((2, NACC, HH), jnp.float32),
        mesh=mesh,
        scratch_types=[
            pltpu.VMEM((IDXB, CH), jnp.int32),
            pltpu.VMEM((IDXB, CH), jnp.int32),
            pltpu.VMEM((NSLOT, CH, HH), jnp.float32),
            pltpu.VMEM_SHARED((NACC, HH), jnp.float32),
            pltpu.SemaphoreType.DMA,
            pltpu.SemaphoreType.DMA,
        ],
        compiler_params=pltpu.CompilerParams(use_tc_tiling_on_sc=False),
    )(table, src2d, dst2d)


def _tc_layer_body(acc_ref, h_ref, wl_ref, wr_ref, g_ref, be_ref, out_ref,
                   z_ref, st_ref):
    p = pl.program_id(0)
    b = pl.program_id(1)

    @pl.when(p == 0)
    def _():
        agg = jnp.concatenate([acc_ref[0], acc_ref[1]], axis=1)
        h = jnp.concatenate([h_ref[0], h_ref[1]], axis=1)
        z = (lax.dot_general(agg, wl_ref[...], (((1,), (1,)), ((), ())),
                             preferred_element_type=jnp.float32)
             + lax.dot_general(h, wr_ref[...], (((1,), (1,)), ((), ())),
                               preferred_element_type=jnp.float32))
        z_ref[pl.ds(b * BLK, BLK), :] = z
        s0 = jnp.sum(z, axis=0, keepdims=True)
        s1 = jnp.sum(z * z, axis=0, keepdims=True)

        @pl.when(b == 0)
        def _():
            st_ref[0:1, :] = s0
            st_ref[1:2, :] = s1

        @pl.when(b > 0)
        def _():
            st_ref[0:1, :] = st_ref[0:1, :] + s0
            st_ref[1:2, :] = st_ref[1:2, :] + s1

    @pl.when(p == 1)
    def _():
        mu = st_ref[0:1, :] * (1.0 / NNODE)
        var = st_ref[1:2, :] * (1.0 / NNODE) - mu * mu
        inv = g_ref[...] * lax.rsqrt(var + BNEPS)
        z = z_ref[pl.ds(b * BLK, BLK), :]
        hn = jnp.tanh(z * inv + (be_ref[...] - mu * inv))
        out_ref[0] = hn[:, :HH]
        out_ref[1] = hn[:, HH:]


def _tc_layer(acc, h, Wl, Wr, gamma, beta):
    return pl.pallas_call(
        _tc_layer_body,
        grid=(2, NB),
        in_specs=[
            pl.BlockSpec((2, BLK, HH), lambda p, b: (0, b, 0)),
            pl.BlockSpec((2, BLK, HH), lambda p, b: (0, b, 0)),
            pl.BlockSpec((HID, HID), lambda p, b: (0, 0)),
            pl.BlockSpec((HID, HID), lambda p, b: (0, 0)),
            pl.BlockSpec((1, HID), lambda p, b: (0, 0)),
            pl.BlockSpec((1, HID), lambda p, b: (0, 0)),
        ],
        out_specs=pl.BlockSpec((2, BLK, HH), lambda p, b: (0, b, 0)),
        out_shape=jax.ShapeDtypeStruct((2, NACC, HH), jnp.float32),
        scratch_shapes=[
            pltpu.VMEM((NNODE, HID), jnp.float32),
            pltpu.VMEM((2, HID), jnp.float32),
        ],
    )(acc, h, Wl, Wr, gamma, beta)


def _tc_head_body(h_ref, wh_ref, bh_ref, out_ref):
    h = jnp.concatenate([h_ref[0], h_ref[1]], axis=1)
    out_ref[...] = lax.dot_general(
        h, wh_ref[...], (((1,), (1,)), ((), ())),
        preferred_element_type=jnp.float32) + bh_ref[...]


def _tc_head(h, Wh, bh):
    return pl.pallas_call(
        _tc_head_body,
        grid=(NB,),
        in_specs=[
            pl.BlockSpec((2, BLK, HH), lambda b: (0, b, 0)),
            pl.BlockSpec((ODIM, HID), lambda b: (0, 0)),
            pl.BlockSpec((1, ODIM), lambda b: (0, 0)),
        ],
        out_specs=pl.BlockSpec((BLK, ODIM), lambda b: (b, 0)),
        out_shape=jax.ShapeDtypeStruct((NNODE, ODIM), jnp.float32),
    )(h, Wh, bh)


def kernel(x, edge_index, Wl0, bl0, Wr0, gamma0, beta0, Wls, bls, Wrs,
           gammas, betas, Wh, bh):
    del bl0, bls
    src = edge_index[0]
    dst = edge_index[1]
    npad = EPAD - NEDGE
    src2d = jnp.concatenate(
        [src, jnp.zeros((npad,), jnp.int32)]).reshape(EROWS, CH)
    dst2d = jnp.concatenate(
        [dst, jnp.full((npad,), NNODE, jnp.int32)]).reshape(EROWS, CH)

    h0 = jnp.zeros((2, NACC, HH), jnp.float32).at[0, :NNODE, :IN_D].set(x)
    wpad = jnp.zeros((HID, HID - IN_D), jnp.float32)
    Wl_all = jnp.concatenate(
        [jnp.concatenate([Wl0, wpad], axis=1)[None], Wls], axis=0)
    Wr_all = jnp.concatenate(
        [jnp.concatenate([Wr0, wpad], axis=1)[None], Wrs], axis=0)
    g_all = jnp.concatenate([gamma0[None], gammas], axis=0)
    b_all = jnp.concatenate([beta0[None], betas], axis=0)

    def layer(h, ws):
        Wl, Wr, g, be = ws
        acc = _sc_agg(h, src2d, dst2d)
        h2 = _tc_layer(acc, h, Wl, Wr, g.reshape(1, HID), be.reshape(1, HID))
        return h2, None

    h, _ = lax.scan(layer, h0, (Wl_all, Wr_all, g_all, b_all))
    return _tc_head(h, Wh, bh.reshape(1, ODIM))
